# baseline (speedup 1.0000x reference)
import functools

import jax
import jax.numpy as jnp
from jax.experimental import pallas as pl
from jax.experimental.pallas import tpu as pltpu
from jax.experimental.pallas import tpu_sc as plsc

TOPK = 100
SCALE = 4.0


def _topk_kernel(hm_ref, out_ref, *scrs, B, C, H, W):
    hm_scrs = scrs[:B]
    rowmax_scrs = scrs[B:2 * B]
    rowmax2_scrs = scrs[2 * B:]
    NROWS = C * H
    s = pl.program_id(0)

    for b in range(B):

        @pl.when(s == b)
        def _maxpool(b=b):
            x = hm_ref[0].reshape(NROWS, W)
            NEG = jnp.float32(-3.0e38)
            yc = jax.lax.broadcasted_iota(jnp.int32, (NROWS, W), 0) % H
            neg_row = jnp.full((1, W), NEG, jnp.float32)
            xm1 = jnp.concatenate([neg_row, x[:-1, :]], axis=0)
            xp1 = jnp.concatenate([x[1:, :], neg_row], axis=0)
            vmax = jnp.maximum(x, jnp.maximum(
                jnp.where(yc == 0, NEG, xm1),
                jnp.where(yc == H - 1, NEG, xp1)))
            neg_col = jnp.full((NROWS, 1), NEG, jnp.float32)
            hl = jnp.concatenate([neg_col, vmax[:, :-1]], axis=1)
            hr = jnp.concatenate([vmax[:, 1:], neg_col], axis=1)
            hmax = jnp.maximum(vmax, jnp.maximum(hl, hr))

            hm = jnp.where(hmax == x, x, jnp.float32(0.0))
            hm_scrs[b][...] = hm
            rowmax_scrs[b][...] = jnp.max(hm.reshape(C, H, W), axis=2)
            lane_b = jax.lax.broadcasted_iota(jnp.int32, (NROWS, W), 1)
            rmax = jnp.max(hm, axis=1, keepdims=True)
            first = jnp.min(jnp.where(hm == rmax, lane_b, W),
                            axis=1, keepdims=True)
            hm2 = jnp.where(lane_b == first, jnp.float32(-1.0), hm)
            rowmax2_scrs[b][...] = jnp.max(hm2.reshape(C, H, W), axis=2)

    @pl.when(s == B)
    def _rounds():
        lane = jax.lax.broadcasted_iota(jnp.int32, (1, W), 1)
        ridx = (jax.lax.broadcasted_iota(jnp.int32, (C, H), 0) * H
                + jax.lax.broadcasted_iota(jnp.int32, (C, H), 1))
        NEGF = jnp.float32(-3.0e38)
        R = range(B)

        def stage1_body(k, carries):
            km = (lane == k)
            rms = [rowmax_scrs[b][...] for b in R]
            ms = [jnp.max(rms[b]) for b in R]
            rids = [jnp.min(jnp.where(rms[b] == ms[b], ridx, NROWS))
                    for b in R]
            secs = [jnp.max(jnp.where(ridx == rids[b], rowmax2_scrs[b][...],
                                      NEGF)) for b in R]
            for b in R:
                rowmax_scrs[b][...] = jnp.where(ridx == rids[b], NEGF, rms[b])
            out = []
            for b in R:
                tbl, tb2, rv = carries[b]
                out.append((
                    jnp.where(km, ms[b], tbl),
                    jnp.where(km, secs[b], tb2),
                    jnp.where(km, rids[b], rv),
                ))
            return tuple(out)

        zf = jnp.full((1, W), NEGF, jnp.float32)
        zi = jnp.zeros((1, W), jnp.int32)
        st1 = jax.lax.fori_loop(0, W, stage1_body,
                                tuple((zf, zf, zi) for _ in R))

        ridvs = [st1[b][2] for b in R]

        def stage2_body(k, carries):
            km = (lane == k)
            new = []
            tbls = [carries[b][0] for b in R]
            tb2s = [carries[b][1] for b in R]
            ms = [jnp.max(tbls[b]) for b in R]
            ridmins = [jnp.min(jnp.where(tbls[b] == ms[b], ridvs[b], NROWS))
                       for b in R]
            kws = [jnp.min(jnp.where((tbls[b] == ms[b])
                                     & (ridvs[b] == ridmins[b]), lane, W),
                           axis=1, keepdims=True) for b in R]
            secs = [jnp.max(jnp.where(lane == kws[b], tb2s[b], NEGF))
                    for b in R]
            tblns = [jnp.where(lane == kws[b], secs[b], tbls[b]) for b in R]

            rscs = ridmins
            rows = [hm_scrs[b][pl.ds(rscs[b], 1), :] for b in R]
            xqs = [jnp.min(jnp.where(rows[b] == ms[b], lane, W),
                           axis=1, keepdims=True) for b in R]
            newrows = [jnp.where(lane == xqs[b], jnp.float32(-1.0), rows[b])
                       for b in R]
            for b in R:
                hm_scrs[b][pl.ds(rscs[b], 1), :] = newrows[b]
            xq2s = [jnp.min(jnp.where(newrows[b] == secs[b], lane, W),
                            axis=1, keepdims=True) for b in R]
            val2s = [jnp.max(jnp.where(lane == xq2s[b], jnp.float32(-1.0),
                                       newrows[b]), axis=1, keepdims=True)
                     for b in R]
            tb2ns = [jnp.where(lane == kws[b], val2s[b], tb2s[b]) for b in R]

            for b in R:
                sc_v, r_v, x_v = carries[b][2:]
                new.append((
                    tblns[b], tb2ns[b],
                    jnp.where(km, ms[b], sc_v),
                    jnp.where(km, ridmins[b].astype(jnp.float32), r_v),
                    jnp.where(km, xqs[b].astype(jnp.float32), x_v),
                ))
            return tuple(new)

        z = jnp.zeros((1, W), jnp.float32)
        init2 = tuple((st1[b][0], st1[b][1], z, z, z) for b in R)
        outs = jax.lax.fori_loop(0, TOPK, stage2_body, init2)
        zz = jnp.zeros((5, W), jnp.float32)
        for b in range(B):
            out_ref[b] = jnp.concatenate([outs[b][2], outs[b][3],
                                          outs[b][4], zz], axis=0)


def _decode_sc(packed_hbm, off_hbm, wh_hbm, out_hbm,
               score_v, r_v, x_v, idx_v, rows_v, o_v, sem, *, H, W):
    cid = jax.lax.axis_index("c")
    sid = jax.lax.axis_index("s")
    wid = sid * 2 + cid
    b = wid // 8
    base = (wid - b * 8) * 16

    pltpu.sync_copy(packed_hbm.at[b, 0, pl.ds(base, 16)], score_v)
    pltpu.sync_copy(packed_hbm.at[b, 1, pl.ds(base, 16)], r_v)
    pltpu.sync_copy(packed_hbm.at[b, 2, pl.ds(base, 16)], x_v)

    rv = r_v[...].astype(jnp.int32)
    xi = x_v[...].astype(jnp.int32)
    ci = rv // H
    yi = rv - ci * H
    idx_v[...] = yi
    cps = [pltpu.async_copy(src.at[b, ch].at[idx_v], rows_v.at[i], sem)
           for i, (src, ch) in enumerate(
               [(off_hbm, 0), (off_hbm, 1), (wh_hbm, 0), (wh_hbm, 1)])]
    for cp in cps:
        cp.wait()

    io = jax.lax.iota(jnp.int32, 16)
    offx = plsc.load_gather(rows_v.at[0], [io, xi])
    offy = plsc.load_gather(rows_v.at[1], [io, xi])
    bw = plsc.load_gather(rows_v.at[2], [io, xi])
    bh = plsc.load_gather(rows_v.at[3], [io, xi])

    xs = xi.astype(jnp.float32) + offx
    ys = yi.astype(jnp.float32) + offy
    o_v[0, :] = ci.astype(jnp.float32)
    o_v[1, :] = score_v[...]
    o_v[2, :] = (xs - 0.5 * bw) * SCALE
    o_v[3, :] = (ys - 0.5 * bh) * SCALE
    o_v[4, :] = (xs + 0.5 * bw) * SCALE
    o_v[5, :] = (ys + 0.5 * bh) * SCALE
    for i in range(6):
        pltpu.sync_copy(o_v.at[i], out_hbm.at[b, i, pl.ds(base, 16)])


def kernel(heatmap, offset, wh):
    B, C, H, W = heatmap.shape
    packed = pl.pallas_call(
        functools.partial(_topk_kernel, B=B, C=C, H=H, W=W),
        grid=(B + 1,),
        in_specs=[
            pl.BlockSpec((1, C, H, W),
                         lambda s: (jnp.minimum(s, B - 1), 0, 0, 0)),
        ],
        out_specs=pl.BlockSpec((B, 8, W), lambda s: (0, 0, 0)),
        out_shape=jax.ShapeDtypeStruct((B, 8, W), jnp.float32),
        scratch_shapes=(
            [pltpu.VMEM((C * H, W), jnp.float32) for _ in range(B)]
            + [pltpu.VMEM((C, H), jnp.float32) for _ in range(B)]
            + [pltpu.VMEM((C, H), jnp.float32) for _ in range(B)]
        ),
    )(heatmap)

    mesh = plsc.VectorSubcoreMesh(core_axis_name="c", subcore_axis_name="s")
    decode = pl.kernel(
        functools.partial(_decode_sc, H=H, W=W),
        out_type=jax.ShapeDtypeStruct((B, 8, W), jnp.float32),
        mesh=mesh,
        scratch_types=[
            pltpu.VMEM((16,), jnp.float32),
            pltpu.VMEM((16,), jnp.float32),
            pltpu.VMEM((16,), jnp.float32),
            pltpu.VMEM((16,), jnp.int32),
            pltpu.VMEM((4, 16, W), jnp.float32),
            pltpu.VMEM((8, 16), jnp.float32),
            pltpu.SemaphoreType.DMA,
        ],
        compiler_params=pltpu.CompilerParams(needs_layout_passes=False,
                                             use_tc_tiling_on_sc=False),
    )
    out = decode(packed, offset, wh)

    ids = out[:, 0, :TOPK][:, :, None]
    scores = out[:, 1, :TOPK][:, :, None]
    bboxes = jnp.transpose(out[:, 2:6, :TOPK], (0, 2, 1))
    return (ids, scores, bboxes)


# speedup vs baseline: 1.9774x; 1.9774x over previous
import functools

import jax
import jax.numpy as jnp
from jax.experimental import pallas as pl
from jax.experimental.pallas import tpu as pltpu
from jax.experimental.pallas import tpu_sc as plsc

TOPK = 100
SCALE = 4.0


def _topk_kernel(hm_ref, out_ref, *scrs, B, C, H, W):
    hm_scrs = scrs[:B]
    rowmax_scrs = scrs[B:]
    NROWS = C * H
    s = pl.program_id(0)

    for b in range(B):

        @pl.when(s == b)
        def _maxpool(b=b):
            x = hm_ref[0].reshape(NROWS, W)
            NEG = jnp.float32(-3.0e38)
            yc = jax.lax.broadcasted_iota(jnp.int32, (NROWS, W), 0) % H
            neg_row = jnp.full((1, W), NEG, jnp.float32)
            xm1 = jnp.concatenate([neg_row, x[:-1, :]], axis=0)
            xp1 = jnp.concatenate([x[1:, :], neg_row], axis=0)
            vmax = jnp.maximum(x, jnp.maximum(
                jnp.where(yc == 0, NEG, xm1),
                jnp.where(yc == H - 1, NEG, xp1)))
            neg_col = jnp.full((NROWS, 1), NEG, jnp.float32)
            hl = jnp.concatenate([neg_col, vmax[:, :-1]], axis=1)
            hr = jnp.concatenate([vmax[:, 1:], neg_col], axis=1)
            hmax = jnp.maximum(vmax, jnp.maximum(hl, hr))

            hm = jnp.where(hmax == x, x, jnp.float32(0.0))
            hm_scrs[b][...] = hm
            rowmax_scrs[b][...] = jnp.max(hm.reshape(C, H, W), axis=2)

    @pl.when(s == B)
    def _rounds():
        lane = jax.lax.broadcasted_iota(jnp.int32, (1, W), 1)
        lane_h = jax.lax.broadcasted_iota(jnp.int32, (1, H), 1)
        ridx = (jax.lax.broadcasted_iota(jnp.int32, (C, H), 0) * H
                + jax.lax.broadcasted_iota(jnp.int32, (C, H), 1))
        R = range(B)

        def round_body(k, carries):
            rms = [rowmax_scrs[b][...] for b in R]
            ms = [jnp.max(rms[b]) for b in R]
            rs = [jnp.min(jnp.where(rms[b] == ms[b], ridx, NROWS)) for b in R]
            rows = [hm_scrs[b][pl.ds(rs[b], 1), :] for b in R]
            xqs = [jnp.min(jnp.where(rows[b] == ms[b], lane, W),
                           axis=1, keepdims=True) for b in R]
            newrows = [jnp.where(lane == xqs[b], jnp.float32(-1.0), rows[b])
                       for b in R]
            for b in R:
                hm_scrs[b][pl.ds(rs[b], 1), :] = newrows[b]
            nrms = [jnp.max(newrows[b], axis=1, keepdims=True) for b in R]
            cs = [rs[b] // H for b in R]
            ys = [rs[b] - cs[b] * H for b in R]
            rmrows = [rowmax_scrs[b][pl.ds(cs[b], 1), :] for b in R]
            for b in R:
                rowmax_scrs[b][pl.ds(cs[b], 1), :] = jnp.where(
                    lane_h == ys[b], nrms[b], rmrows[b])

            km = (lane == k)
            out = []
            for b in R:
                sc_v, r_v, x_v = carries[b]
                out.append((
                    jnp.where(km, ms[b], sc_v),
                    jnp.where(km, rs[b].astype(jnp.float32), r_v),
                    jnp.where(km, xqs[b].astype(jnp.float32), x_v),
                ))
            return tuple(out)

        z = jnp.zeros((1, W), jnp.float32)
        init = tuple((z, z, z) for _ in range(B))
        outs = jax.lax.fori_loop(0, TOPK, round_body, init, unroll=4)
        zz = jnp.zeros((5, W), jnp.float32)
        for b in range(B):
            out_ref[b] = jnp.concatenate(list(outs[b]) + [zz], axis=0)


def _decode_sc(packed_hbm, off_hbm, wh_hbm, out_hbm,
               score_v, r_v, x_v, idx_v, rows_v, o_v, sem, *, H, W):
    cid = jax.lax.axis_index("c")
    sid = jax.lax.axis_index("s")
    wid = sid * 2 + cid
    b = wid // 8
    base = (wid - b * 8) * 16

    pltpu.sync_copy(packed_hbm.at[b, 0, pl.ds(base, 16)], score_v)
    pltpu.sync_copy(packed_hbm.at[b, 1, pl.ds(base, 16)], r_v)
    pltpu.sync_copy(packed_hbm.at[b, 2, pl.ds(base, 16)], x_v)

    rv = r_v[...].astype(jnp.int32)
    xi = x_v[...].astype(jnp.int32)
    ci = rv // H
    yi = rv - ci * H
    idx_v[...] = yi
    cps = [pltpu.async_copy(src.at[b, ch].at[idx_v], rows_v.at[i], sem)
           for i, (src, ch) in enumerate(
               [(off_hbm, 0), (off_hbm, 1), (wh_hbm, 0), (wh_hbm, 1)])]
    for cp in cps:
        cp.wait()

    io = jax.lax.iota(jnp.int32, 16)
    offx = plsc.load_gather(rows_v.at[0], [io, xi])
    offy = plsc.load_gather(rows_v.at[1], [io, xi])
    bw = plsc.load_gather(rows_v.at[2], [io, xi])
    bh = plsc.load_gather(rows_v.at[3], [io, xi])

    xs = xi.astype(jnp.float32) + offx
    ys = yi.astype(jnp.float32) + offy
    o_v[0, :] = ci.astype(jnp.float32)
    o_v[1, :] = score_v[...]
    o_v[2, :] = (xs - 0.5 * bw) * SCALE
    o_v[3, :] = (ys - 0.5 * bh) * SCALE
    o_v[4, :] = (xs + 0.5 * bw) * SCALE
    o_v[5, :] = (ys + 0.5 * bh) * SCALE
    for i in range(6):
        pltpu.sync_copy(o_v.at[i], out_hbm.at[b, i, pl.ds(base, 16)])


def kernel(heatmap, offset, wh):
    B, C, H, W = heatmap.shape
    packed = pl.pallas_call(
        functools.partial(_topk_kernel, B=B, C=C, H=H, W=W),
        grid=(B + 1,),
        in_specs=[
            pl.BlockSpec((1, C, H, W),
                         lambda s: (jnp.minimum(s, B - 1), 0, 0, 0)),
        ],
        out_specs=pl.BlockSpec((B, 8, W), lambda s: (0, 0, 0)),
        out_shape=jax.ShapeDtypeStruct((B, 8, W), jnp.float32),
        scratch_shapes=(
            [pltpu.VMEM((C * H, W), jnp.float32) for _ in range(B)]
            + [pltpu.VMEM((C, H), jnp.float32) for _ in range(B)]
        ),
    )(heatmap)

    mesh = plsc.VectorSubcoreMesh(core_axis_name="c", subcore_axis_name="s")
    decode = pl.kernel(
        functools.partial(_decode_sc, H=H, W=W),
        out_type=jax.ShapeDtypeStruct((B, 8, W), jnp.float32),
        mesh=mesh,
        scratch_types=[
            pltpu.VMEM((16,), jnp.float32),
            pltpu.VMEM((16,), jnp.float32),
            pltpu.VMEM((16,), jnp.float32),
            pltpu.VMEM((16,), jnp.int32),
            pltpu.VMEM((4, 16, W), jnp.float32),
            pltpu.VMEM((8, 16), jnp.float32),
            pltpu.SemaphoreType.DMA,
        ],
        compiler_params=pltpu.CompilerParams(needs_layout_passes=False,
                                             use_tc_tiling_on_sc=False),
    )
    out = decode(packed, offset, wh)

    ids = out[:, 0, :TOPK][:, :, None]
    scores = out[:, 1, :TOPK][:, :, None]
    bboxes = jnp.transpose(out[:, 2:6, :TOPK], (0, 2, 1))
    return (ids, scores, bboxes)


# speedup vs baseline: 1.9799x; 1.0012x over previous
import functools

import jax
import jax.numpy as jnp
from jax.experimental import pallas as pl
from jax.experimental.pallas import tpu as pltpu
from jax.experimental.pallas import tpu_sc as plsc

TOPK = 100
SCALE = 4.0


def _topk_kernel(hm_ref, out_ref, *scrs, B, C, H, W):
    hm_scrs = scrs[:B]
    rowmax_scrs = scrs[B:]
    NROWS = C * H
    s = pl.program_id(0)

    for b in range(B):

        @pl.when(s == b)
        def _maxpool(b=b):
            x = hm_ref[0].reshape(NROWS, W)
            NEG = jnp.float32(-3.0e38)
            yc = jax.lax.broadcasted_iota(jnp.int32, (NROWS, W), 0) % H
            ln = jax.lax.broadcasted_iota(jnp.int32, (NROWS, W), 1)
            xm1 = pltpu.roll(x, 1, 0)
            xp1 = pltpu.roll(x, NROWS - 1, 0)
            vmax = jnp.maximum(x, jnp.maximum(
                jnp.where(yc == 0, NEG, xm1),
                jnp.where(yc == H - 1, NEG, xp1)))
            hl = pltpu.roll(vmax, 1, 1)
            hr = pltpu.roll(vmax, W - 1, 1)
            hmax = jnp.maximum(vmax, jnp.maximum(
                jnp.where(ln == 0, NEG, hl),
                jnp.where(ln == W - 1, NEG, hr)))

            hm = jnp.where(hmax == x, x, jnp.float32(0.0))
            hm_scrs[b][...] = hm
            rowmax_scrs[b][...] = jnp.max(hm.reshape(C, H, W), axis=2)

    @pl.when(s == B)
    def _rounds():
        lane = jax.lax.broadcasted_iota(jnp.int32, (1, W), 1)
        lane_h = jax.lax.broadcasted_iota(jnp.int32, (1, H), 1)
        ridx = (jax.lax.broadcasted_iota(jnp.int32, (C, H), 0) * H
                + jax.lax.broadcasted_iota(jnp.int32, (C, H), 1))
        R = range(B)

        def round_body(k, carries):
            rms = [rowmax_scrs[b][...] for b in R]
            ms = [jnp.max(rms[b]) for b in R]
            rs = [jnp.min(jnp.where(rms[b] == ms[b], ridx, NROWS)) for b in R]
            rows = [hm_scrs[b][pl.ds(rs[b], 1), :] for b in R]
            xqs = [jnp.min(jnp.where(rows[b] == ms[b], lane, W),
                           axis=1, keepdims=True) for b in R]
            newrows = [jnp.where(lane == xqs[b], jnp.float32(-1.0), rows[b])
                       for b in R]
            for b in R:
                hm_scrs[b][pl.ds(rs[b], 1), :] = newrows[b]
            nrms = [jnp.max(newrows[b], axis=1, keepdims=True) for b in R]
            cs = [rs[b] // H for b in R]
            ys = [rs[b] - cs[b] * H for b in R]
            rmrows = [rowmax_scrs[b][pl.ds(cs[b], 1), :] for b in R]
            for b in R:
                rowmax_scrs[b][pl.ds(cs[b], 1), :] = jnp.where(
                    lane_h == ys[b], nrms[b], rmrows[b])

            km = (lane == k)
            out = []
            for b in R:
                sc_v, r_v, x_v = carries[b]
                out.append((
                    jnp.where(km, ms[b], sc_v),
                    jnp.where(km, rs[b].astype(jnp.float32), r_v),
                    jnp.where(km, xqs[b].astype(jnp.float32), x_v),
                ))
            return tuple(out)

        z = jnp.zeros((1, W), jnp.float32)
        init = tuple((z, z, z) for _ in range(B))
        outs = jax.lax.fori_loop(0, TOPK, round_body, init, unroll=4)
        zz = jnp.zeros((5, W), jnp.float32)
        for b in range(B):
            out_ref[b] = jnp.concatenate(list(outs[b]) + [zz], axis=0)


def _decode_sc(packed_hbm, off_hbm, wh_hbm, out_hbm,
               score_v, r_v, x_v, idx_v, rows_v, o_v, sem, *, H, W):
    cid = jax.lax.axis_index("c")
    sid = jax.lax.axis_index("s")
    wid = sid * 2 + cid
    b = wid // 8
    base = (wid - b * 8) * 16

    pltpu.sync_copy(packed_hbm.at[b, 0, pl.ds(base, 16)], score_v)
    pltpu.sync_copy(packed_hbm.at[b, 1, pl.ds(base, 16)], r_v)
    pltpu.sync_copy(packed_hbm.at[b, 2, pl.ds(base, 16)], x_v)

    rv = r_v[...].astype(jnp.int32)
    xi = x_v[...].astype(jnp.int32)
    ci = rv // H
    yi = rv - ci * H
    idx_v[...] = yi
    cps = [pltpu.async_copy(src.at[b, ch].at[idx_v], rows_v.at[i], sem)
           for i, (src, ch) in enumerate(
               [(off_hbm, 0), (off_hbm, 1), (wh_hbm, 0), (wh_hbm, 1)])]
    for cp in cps:
        cp.wait()

    io = jax.lax.iota(jnp.int32, 16)
    offx = plsc.load_gather(rows_v.at[0], [io, xi])
    offy = plsc.load_gather(rows_v.at[1], [io, xi])
    bw = plsc.load_gather(rows_v.at[2], [io, xi])
    bh = plsc.load_gather(rows_v.at[3], [io, xi])

    xs = xi.astype(jnp.float32) + offx
    ys = yi.astype(jnp.float32) + offy
    o_v[0, :] = ci.astype(jnp.float32)
    o_v[1, :] = score_v[...]
    o_v[2, :] = (xs - 0.5 * bw) * SCALE
    o_v[3, :] = (ys - 0.5 * bh) * SCALE
    o_v[4, :] = (xs + 0.5 * bw) * SCALE
    o_v[5, :] = (ys + 0.5 * bh) * SCALE
    for i in range(6):
        pltpu.sync_copy(o_v.at[i], out_hbm.at[b, i, pl.ds(base, 16)])


def kernel(heatmap, offset, wh):
    B, C, H, W = heatmap.shape
    packed = pl.pallas_call(
        functools.partial(_topk_kernel, B=B, C=C, H=H, W=W),
        grid=(B + 1,),
        in_specs=[
            pl.BlockSpec((1, C, H, W),
                         lambda s: (jnp.minimum(s, B - 1), 0, 0, 0)),
        ],
        out_specs=pl.BlockSpec((B, 8, W), lambda s: (0, 0, 0)),
        out_shape=jax.ShapeDtypeStruct((B, 8, W), jnp.float32),
        scratch_shapes=(
            [pltpu.VMEM((C * H, W), jnp.float32) for _ in range(B)]
            + [pltpu.VMEM((C, H), jnp.float32) for _ in range(B)]
        ),
    )(heatmap)

    mesh = plsc.VectorSubcoreMesh(core_axis_name="c", subcore_axis_name="s")
    decode = pl.kernel(
        functools.partial(_decode_sc, H=H, W=W),
        out_type=jax.ShapeDtypeStruct((B, 8, W), jnp.float32),
        mesh=mesh,
        scratch_types=[
            pltpu.VMEM((16,), jnp.float32),
            pltpu.VMEM((16,), jnp.float32),
            pltpu.VMEM((16,), jnp.float32),
            pltpu.VMEM((16,), jnp.int32),
            pltpu.VMEM((4, 16, W), jnp.float32),
            pltpu.VMEM((8, 16), jnp.float32),
            pltpu.SemaphoreType.DMA,
        ],
        compiler_params=pltpu.CompilerParams(needs_layout_passes=False,
                                             use_tc_tiling_on_sc=False),
    )
    out = decode(packed, offset, wh)

    ids = out[:, 0, :TOPK][:, :, None]
    scores = out[:, 1, :TOPK][:, :, None]
    bboxes = jnp.transpose(out[:, 2:6, :TOPK], (0, 2, 1))
    return (ids, scores, bboxes)
